# baseline (speedup 1.0000x reference)
import jax
import jax.numpy as jnp
from jax import lax
from jax.experimental import pallas as pl
from jax.experimental.pallas import tpu as pltpu
from jax.experimental.pallas import tpu_sc as plsc

VOCAB = 9487
BATCH = 4096
SEQ = 20
N_IMG = 1024
MAX_CAPS = 5

NW = 32
ROWS_W = BATCH // NW
IMGS_W = N_IMG // NW
SEQ_EL = ROWS_W * SEQ
GT_EL = IMGS_W * MAX_CAPS * SEQ
TOK_PAD = 9504
ST_PAD = 1024
TOK_W = TOK_PAD // 2
ST_W = ST_PAD // 2
TAB_W = TOK_W + ST_W
GT_W = GT_EL // 2
SEQ_W = SEQ_EL // 2
SCO_W = 256


def _halves_i32(xi):
    return xi & 0xFFFF, lax.shift_right_logical(xi, 16)


def _halves_f32(xi):
    lo = plsc.bitcast(lax.shift_left(xi, 16), jnp.float32)
    hi = plsc.bitcast(xi & jnp.int32(-65536), jnp.float32)
    return lo, hi


def _sc_body(tab, gtnc, gen3, gre3, ncap_h, sco_out,
             tab_v, gtnc_v, gen_v, gre_v,
             tok_f, st_i, tsm_v, cnt_v, gsm_v, gcn_v,
             caps_v, gimg_v, rs_v, sco_v, ncap_v,
             sem0, sem1, sem2, sem3, sem4):
    wid = lax.axis_index("s") * 2 + lax.axis_index("c")

    cp_tab = pltpu.async_copy(tab, tab_v, sem0)
    cp_gtn = pltpu.async_copy(
        gtnc.at[pl.ds(pl.multiple_of(wid * GT_W, 8), GT_W)], gtnc_v, sem1)
    cp_gen = pltpu.async_copy(
        gen3.at[pl.ds(pl.multiple_of(wid * 3 * SEQ_W, 8), 3 * SEQ_W)],
        gen_v, sem2)
    cp_gre = pltpu.async_copy(
        gre3.at[pl.ds(pl.multiple_of(wid * 3 * SEQ_W, 8), 3 * SEQ_W)],
        gre_v, sem3)
    cp_nc = pltpu.async_copy(
        ncap_h.at[pl.ds(pl.multiple_of(wid * IMGS_W, 8), IMGS_W)],
        ncap_v, sem4)

    iota = lax.iota(jnp.int32, 16)
    iota20 = iota * SEQ
    zero16 = jnp.zeros((16,), jnp.float32)

    cp_tab.wait()

    @pl.loop(0, TOK_W // 16)
    def _(i):
        a, b = _halves_f32(tab_v[pl.ds(pl.multiple_of(i * 16, 16), 16)])
        tok_f[pl.ds(pl.multiple_of(i * 32, 32), 16)] = a
        tok_f[pl.ds(pl.multiple_of(i * 32 + 16, 16), 16)] = b

    @pl.loop(0, ST_W // 16)
    def _(i):
        a, b = _halves_i32(tab_v[pl.ds(pl.multiple_of(TOK_W + i * 16, 16), 16)])
        st_i[pl.ds(pl.multiple_of(i * 32, 32), 16)] = a
        st_i[pl.ds(pl.multiple_of(i * 32 + 16, 16), 16)] = b

    cp_gtn.wait()

    @pl.loop(0, GT_W // 16)
    def _(k):
        t0, t1 = _halves_i32(gtnc_v[pl.ds(pl.multiple_of(k * 16, 16), 16)])
        for h, tid in ((0, t0), (1, t1)):
            ts = plsc.load_gather(tok_f, [jnp.minimum(tid, TOK_PAD - 1)])
            valid = tid != 0
            off = pl.multiple_of(k * 32 + h * 16, 16)
            gsm_v[pl.ds(off, 16)] = jnp.where(valid, ts, 0.0)
            gcn_v[pl.ds(off, 16)] = jnp.where(valid, 1.0, 0.0)

    @pl.loop(0, (IMGS_W * MAX_CAPS) // 16)
    def _(g):
        def t_body(t, carry):
            s, c = carry
            idx = iota20 + (g * 320 + t)
            s = s + plsc.load_gather(gsm_v, [idx])
            c = c + plsc.load_gather(gcn_v, [idx])
            return s, c

        s, c = lax.fori_loop(0, SEQ, t_body, (zero16, zero16))
        caps_v[pl.ds(pl.multiple_of(g * 16, 16), 16)] = s / jnp.maximum(c, 1.0)

    cp_nc.wait()
    nc0 = ncap_v[pl.ds(0, 16)]
    nc1 = ncap_v[pl.ds(16, 16)]
    for grp, ncg in ((0, nc0), (1, nc1)):
        imgoff = (grp * 16 + iota) * MAX_CAPS

        def c_body(c, gsum):
            cap = plsc.load_gather(caps_v, [imgoff + c])
            return gsum + jnp.where(c < ncg, cap, 0.0)

        gsum = lax.fori_loop(0, MAX_CAPS, c_body, zero16)
        gimg_v[pl.ds(grp * 16, 16)] = gsum / ncg.astype(jnp.float32)

    def seq_pass(src_v, out_base):
        @pl.loop(0, SEQ_W // 16)
        def _(k):
            tx0, tx1 = _halves_i32(
                src_v[pl.ds(pl.multiple_of(k * 16, 16), 16)])
            bn0, bn1 = _halves_i32(
                src_v[pl.ds(pl.multiple_of(SEQ_W + k * 16, 16), 16)])
            vi0, vi1 = _halves_i32(
                src_v[pl.ds(pl.multiple_of(2 * SEQ_W + k * 16, 16), 16)])
            for h, tx, bn, vi in ((0, tx0, bn0, vi0), (1, tx1, bn1, vi1)):
                mapped = plsc.load_gather(
                    st_i, [jnp.minimum(vi * 2 + bn - 1, ST_PAD - 1)])
                res = jnp.where(tx < VOCAB, tx, mapped)
                ts = plsc.load_gather(tok_f, [jnp.minimum(res, TOK_PAD - 1)])
                valid = res != 0
                off = pl.multiple_of(out_base + k * 32 + h * 16, 16)
                tsm_v[pl.ds(off, 16)] = jnp.where(valid, ts, 0.0)
                cnt_v[pl.ds(off, 16)] = jnp.where(valid, 1.0, 0.0)

    cp_gen.wait()
    seq_pass(gen_v, 0)
    cp_gre.wait()
    seq_pass(gre_v, SEQ_EL)

    @pl.loop(0, (2 * ROWS_W) // 16)
    def _(g):
        def t_body(t, carry):
            s, c = carry
            idx = iota20 + (g * 320 + t)
            s = s + plsc.load_gather(tsm_v, [idx])
            c = c + plsc.load_gather(cnt_v, [idx])
            return s, c

        s, c = lax.fori_loop(0, SEQ, t_body, (zero16, zero16))
        rs_v[pl.ds(pl.multiple_of(g * 16, 16), 16)] = s / jnp.maximum(c, 1.0)

    iota4 = iota // 4
    for g in range(ROWS_W // 16):
        gen_s = rs_v[pl.ds(g * 16, 16)]
        gre_s = rs_v[pl.ds(ROWS_W + g * 16, 16)]
        gtv = plsc.load_gather(gimg_v, [iota4 + g * 4])
        sco_v[pl.ds(g * 16, 16)] = (gen_s - gre_s) * gtv

    pltpu.async_copy(sco_v, sco_out.at[pl.ds(pl.multiple_of(wid * SCO_W, 256),
                                             ROWS_W)], sem0).wait()


def _tc_body(s_ref, rew_ref, mean_ref):
    s = s_ref[...]
    rew_ref[...] = jnp.broadcast_to(s, (BATCH, SEQ))
    mean_ref[...] = jnp.full((1, 1), jnp.sum(s) * (1.0 / BATCH), jnp.float32)


def _perm(x):
    return x.reshape(-1, 2, 16).swapaxes(1, 2).reshape(-1)


def _pack(x_i16):
    return lax.bitcast_convert_type(x_i16.reshape(-1, 2), jnp.int32)


@jax.jit
def kernel(gen_txt_seq, gen_bn_seq, gen_vis_seq, greedy_txt_seq,
           greedy_bn_seq, greedy_vis_seq, gt_gts, ncap, st2towidx,
           token_scores):
    i16 = jnp.int16

    tok_bf = jnp.pad(token_scores.astype(jnp.bfloat16),
                     (0, TOK_PAD - token_scores.shape[0]))
    tok_16 = _perm(lax.bitcast_convert_type(tok_bf, i16))
    st_16 = _perm(jnp.pad(st2towidx.astype(i16),
                          (0, ST_PAD - st2towidx.shape[0])))
    tab = _pack(jnp.concatenate([tok_16, st_16]))

    gtnc = _pack(_perm(gt_gts.astype(i16).reshape(-1)))

    def seq3(a, b, c):
        return _pack(jnp.concatenate(
            [_perm(x.astype(i16).reshape(-1)).reshape(NW, SEQ_EL)
             for x in (a, b, c)], axis=1).reshape(-1))

    gen3 = seq3(gen_txt_seq, gen_bn_seq, gen_vis_seq)
    gre3 = seq3(greedy_txt_seq, greedy_bn_seq, greedy_vis_seq)

    mesh = plsc.VectorSubcoreMesh(core_axis_name="c", subcore_axis_name="s",
                                  num_cores=2, num_subcores=16)
    sc = pl.kernel(
        _sc_body,
        out_type=jax.ShapeDtypeStruct((NW * SCO_W,), jnp.float32),
        mesh=mesh,
        compiler_params=pltpu.CompilerParams(needs_layout_passes=False),
        scratch_types=[
            pltpu.VMEM((TAB_W,), jnp.int32),
            pltpu.VMEM((GT_W,), jnp.int32),
            pltpu.VMEM((3 * SEQ_W,), jnp.int32),
            pltpu.VMEM((3 * SEQ_W,), jnp.int32),
            pltpu.VMEM((TOK_PAD,), jnp.float32),
            pltpu.VMEM((ST_PAD,), jnp.int32),
            pltpu.VMEM((2 * SEQ_EL,), jnp.float32),
            pltpu.VMEM((2 * SEQ_EL,), jnp.float32),
            pltpu.VMEM((GT_EL,), jnp.float32),
            pltpu.VMEM((GT_EL,), jnp.float32),
            pltpu.VMEM((IMGS_W * MAX_CAPS,), jnp.float32),
            pltpu.VMEM((IMGS_W,), jnp.float32),
            pltpu.VMEM((2 * ROWS_W,), jnp.float32),
            pltpu.VMEM((ROWS_W,), jnp.float32),
            pltpu.VMEM((IMGS_W,), jnp.int32),
            pltpu.SemaphoreType.DMA,
            pltpu.SemaphoreType.DMA,
            pltpu.SemaphoreType.DMA,
            pltpu.SemaphoreType.DMA,
            pltpu.SemaphoreType.DMA,
        ],
    )
    scores = sc(tab, gtnc, gen3, gre3,
                ncap.astype(jnp.int32)).reshape(NW, SCO_W)[:, :ROWS_W]

    rewards, mean_arr = pl.pallas_call(
        _tc_body,
        out_shape=(jax.ShapeDtypeStruct((BATCH, SEQ), jnp.float32),
                   jax.ShapeDtypeStruct((1, 1), jnp.float32)),
    )(scores.reshape(BATCH, 1))

    return rewards, mean_arr[0, 0]


# speedup vs baseline: 3.5233x; 3.5233x over previous
import jax
import jax.numpy as jnp
from jax import lax
from jax.experimental import pallas as pl
from jax.experimental.pallas import tpu as pltpu
from jax.experimental.pallas import tpu_sc as plsc

VOCAB = 9487
BATCH = 4096
SEQ = 20
N_IMG = 1024
MAX_CAPS = 5

NW = 32
ROWS_W = BATCH // NW
IMGS_W = N_IMG // NW
SEQ_EL = ROWS_W * SEQ
GT_EL = IMGS_W * MAX_CAPS * SEQ
TOK_PAD = 9504
ST_PAD = 1024
TOK_W = TOK_PAD // 2
ST_W = ST_PAD // 2
TAB_W = TOK_W + ST_W
GT_W = GT_EL // 2
SEQ_W = SEQ_EL // 2
SCO_W = 256


def _halves_i32(xi):
    return xi & 0xFFFF, lax.shift_right_logical(xi, 16)


def _halves_f32(xi):
    lo = plsc.bitcast(lax.shift_left(xi, 16), jnp.float32)
    hi = plsc.bitcast(xi & jnp.int32(-65536), jnp.float32)
    return lo, hi


def _sc_body(tab, gtnc, gen3, gre3, ncap_h, sco_out,
             tab_v, gtnc_v, gen_v, gre_v,
             tok_f, st_i, tsm_v, cnt_v, gsm_v, gcn_v,
             caps_v, gimg_v, rs_v, sco_v, ncap_v,
             sem0, sem1, sem2, sem3, sem4):
    wid = lax.axis_index("s") * 2 + lax.axis_index("c")

    cp_tab = pltpu.async_copy(tab, tab_v, sem0)
    cp_gtn = pltpu.async_copy(
        gtnc.at[pl.ds(pl.multiple_of(wid * GT_W, 8), GT_W)], gtnc_v, sem1)
    cp_gen = pltpu.async_copy(
        gen3.at[pl.ds(pl.multiple_of(wid * 3 * SEQ_W, 8), 3 * SEQ_W)],
        gen_v, sem2)
    cp_gre = pltpu.async_copy(
        gre3.at[pl.ds(pl.multiple_of(wid * 3 * SEQ_W, 8), 3 * SEQ_W)],
        gre_v, sem3)
    cp_nc = pltpu.async_copy(
        ncap_h.at[pl.ds(pl.multiple_of(wid * IMGS_W, 8), IMGS_W)],
        ncap_v, sem4)

    iota = lax.iota(jnp.int32, 16)
    iota20 = iota * SEQ
    zero16 = jnp.zeros((16,), jnp.float32)

    cp_tab.wait()

    @pl.loop(0, TOK_W // 16)
    def _(i):
        a, b = _halves_f32(tab_v[pl.ds(pl.multiple_of(i * 16, 16), 16)])
        tok_f[pl.ds(pl.multiple_of(i * 32, 32), 16)] = a
        tok_f[pl.ds(pl.multiple_of(i * 32 + 16, 16), 16)] = b

    @pl.loop(0, ST_W // 16)
    def _(i):
        a, b = _halves_i32(tab_v[pl.ds(pl.multiple_of(TOK_W + i * 16, 16), 16)])
        st_i[pl.ds(pl.multiple_of(i * 32, 32), 16)] = a
        st_i[pl.ds(pl.multiple_of(i * 32 + 16, 16), 16)] = b

    cp_gtn.wait()

    @pl.loop(0, GT_W // 16)
    def _(k):
        t0, t1 = _halves_i32(gtnc_v[pl.ds(pl.multiple_of(k * 16, 16), 16)])
        for h, tid in ((0, t0), (1, t1)):
            ts = plsc.load_gather(tok_f, [jnp.minimum(tid, TOK_PAD - 1)])
            valid = tid != 0
            off = pl.multiple_of(k * 32 + h * 16, 16)
            gsm_v[pl.ds(off, 16)] = jnp.where(valid, ts, 0.0)
            gcn_v[pl.ds(off, 16)] = jnp.where(valid, 1.0, 0.0)

    @pl.loop(0, (IMGS_W * MAX_CAPS) // 16)
    def _(g):
        def t_body(t, carry):
            s, c = carry
            idx = iota20 + (g * 320 + t)
            s = s + plsc.load_gather(gsm_v, [idx])
            c = c + plsc.load_gather(gcn_v, [idx])
            return s, c

        s, c = lax.fori_loop(0, SEQ, t_body, (zero16, zero16))
        caps_v[pl.ds(pl.multiple_of(g * 16, 16), 16)] = s / jnp.maximum(c, 1.0)

    cp_nc.wait()
    nc0 = ncap_v[pl.ds(0, 16)]
    nc1 = ncap_v[pl.ds(16, 16)]
    for grp, ncg in ((0, nc0), (1, nc1)):
        imgoff = (grp * 16 + iota) * MAX_CAPS

        def c_body(c, gsum):
            cap = plsc.load_gather(caps_v, [imgoff + c])
            return gsum + jnp.where(c < ncg, cap, 0.0)

        gsum = lax.fori_loop(0, MAX_CAPS, c_body, zero16)
        gimg_v[pl.ds(grp * 16, 16)] = gsum / ncg.astype(jnp.float32)

    def seq_pass(src_v, out_base):
        @pl.loop(0, SEQ_W // 16)
        def _(k):
            tx0, tx1 = _halves_i32(
                src_v[pl.ds(pl.multiple_of(k * 16, 16), 16)])
            bn0, bn1 = _halves_i32(
                src_v[pl.ds(pl.multiple_of(SEQ_W + k * 16, 16), 16)])
            vi0, vi1 = _halves_i32(
                src_v[pl.ds(pl.multiple_of(2 * SEQ_W + k * 16, 16), 16)])
            for h, tx, bn, vi in ((0, tx0, bn0, vi0), (1, tx1, bn1, vi1)):
                mapped = plsc.load_gather(
                    st_i, [jnp.minimum(vi * 2 + bn - 1, ST_PAD - 1)])
                res = jnp.where(tx < VOCAB, tx, mapped)
                ts = plsc.load_gather(tok_f, [jnp.minimum(res, TOK_PAD - 1)])
                valid = res != 0
                off = pl.multiple_of(out_base + k * 32 + h * 16, 16)
                tsm_v[pl.ds(off, 16)] = jnp.where(valid, ts, 0.0)
                cnt_v[pl.ds(off, 16)] = jnp.where(valid, 1.0, 0.0)

    cp_gen.wait()
    seq_pass(gen_v, 0)
    cp_gre.wait()
    seq_pass(gre_v, SEQ_EL)

    @pl.loop(0, (2 * ROWS_W) // 16)
    def _(g):
        def t_body(t, carry):
            s, c = carry
            idx = iota20 + (g * 320 + t)
            s = s + plsc.load_gather(tsm_v, [idx])
            c = c + plsc.load_gather(cnt_v, [idx])
            return s, c

        s, c = lax.fori_loop(0, SEQ, t_body, (zero16, zero16))
        rs_v[pl.ds(pl.multiple_of(g * 16, 16), 16)] = s / jnp.maximum(c, 1.0)

    iota4 = iota // 4
    for g in range(ROWS_W // 16):
        gen_s = rs_v[pl.ds(g * 16, 16)]
        gre_s = rs_v[pl.ds(ROWS_W + g * 16, 16)]
        gtv = plsc.load_gather(gimg_v, [iota4 + g * 4])
        sco_v[pl.ds(g * 16, 16)] = (gen_s - gre_s) * gtv

    pltpu.async_copy(sco_v, sco_out.at[pl.ds(pl.multiple_of(wid * SCO_W, 256),
                                             ROWS_W)], sem0).wait()


def _tc_body(s_ref, rew_ref, mean_ref):
    s = s_ref[...]
    rew_ref[...] = jnp.broadcast_to(s, (BATCH, SEQ))
    mean_ref[...] = jnp.full((1, 1), jnp.sum(s) * (1.0 / BATCH), jnp.float32)


def _packw(x):
    v = x.reshape(-1, 2, 16)
    return (v[:, 0, :] | (v[:, 1, :] << 16)).reshape(-1)


@jax.jit
def kernel(gen_txt_seq, gen_bn_seq, gen_vis_seq, greedy_txt_seq,
           greedy_bn_seq, greedy_vis_seq, gt_gts, ncap, st2towidx,
           token_scores):
    i32 = jnp.int32

    ti = lax.bitcast_convert_type(token_scores.astype(jnp.float32), i32)
    th = lax.shift_right_logical(ti + 0x8000, 16)
    tok_h = jnp.pad(th, (0, TOK_PAD - th.shape[0]))
    st_h = jnp.pad(st2towidx.astype(i32), (0, ST_PAD - st2towidx.shape[0]))
    tab = jnp.concatenate([_packw(tok_h), _packw(st_h)])

    gtnc = _packw(gt_gts.astype(i32).reshape(-1))

    def seq3(a, b, c):
        return jnp.concatenate(
            [_packw(x.astype(i32).reshape(-1)).reshape(NW, SEQ_W)
             for x in (a, b, c)], axis=1).reshape(-1)

    gen3 = seq3(gen_txt_seq, gen_bn_seq, gen_vis_seq)
    gre3 = seq3(greedy_txt_seq, greedy_bn_seq, greedy_vis_seq)

    mesh = plsc.VectorSubcoreMesh(core_axis_name="c", subcore_axis_name="s",
                                  num_cores=2, num_subcores=16)
    sc = pl.kernel(
        _sc_body,
        out_type=jax.ShapeDtypeStruct((NW * SCO_W,), jnp.float32),
        mesh=mesh,
        compiler_params=pltpu.CompilerParams(needs_layout_passes=False),
        scratch_types=[
            pltpu.VMEM((TAB_W,), jnp.int32),
            pltpu.VMEM((GT_W,), jnp.int32),
            pltpu.VMEM((3 * SEQ_W,), jnp.int32),
            pltpu.VMEM((3 * SEQ_W,), jnp.int32),
            pltpu.VMEM((TOK_PAD,), jnp.float32),
            pltpu.VMEM((ST_PAD,), jnp.int32),
            pltpu.VMEM((2 * SEQ_EL,), jnp.float32),
            pltpu.VMEM((2 * SEQ_EL,), jnp.float32),
            pltpu.VMEM((GT_EL,), jnp.float32),
            pltpu.VMEM((GT_EL,), jnp.float32),
            pltpu.VMEM((IMGS_W * MAX_CAPS,), jnp.float32),
            pltpu.VMEM((IMGS_W,), jnp.float32),
            pltpu.VMEM((2 * ROWS_W,), jnp.float32),
            pltpu.VMEM((ROWS_W,), jnp.float32),
            pltpu.VMEM((IMGS_W,), jnp.int32),
            pltpu.SemaphoreType.DMA,
            pltpu.SemaphoreType.DMA,
            pltpu.SemaphoreType.DMA,
            pltpu.SemaphoreType.DMA,
            pltpu.SemaphoreType.DMA,
        ],
    )
    scores = sc(tab, gtnc, gen3, gre3,
                ncap.astype(jnp.int32)).reshape(NW, SCO_W)[:, :ROWS_W]

    rewards, mean_arr = pl.pallas_call(
        _tc_body,
        out_shape=(jax.ShapeDtypeStruct((BATCH, SEQ), jnp.float32),
                   jax.ShapeDtypeStruct((1, 1), jnp.float32)),
    )(scores.reshape(BATCH, 1))

    return rewards, mean_arr[0, 0]


# speedup vs baseline: 8.1876x; 2.3238x over previous
import jax
import jax.numpy as jnp
from jax import lax
from jax.experimental import pallas as pl
from jax.experimental.pallas import tpu as pltpu
from jax.experimental.pallas import tpu_sc as plsc

VOCAB = 9487
BATCH = 4096
SEQ = 20
N_IMG = 1024
MAX_CAPS = 5

NW = 32
ROWS_W = BATCH // NW
IMGS_W = N_IMG // NW
SEQ_EL = ROWS_W * SEQ
GT_EL = IMGS_W * MAX_CAPS * SEQ
TOK_PAD = 9728
ST_PAD = 1024
GT_PAD = 3328
TOK_W = TOK_PAD // 2
ST_W = ST_PAD // 2
TAB_W = TOK_W + ST_W
GT_W = GT_PAD // 2
SEQ_W = SEQ_EL // 2
SCO_W = 256


def _halves_i32(xi):
    return xi & 0xFFFF, lax.shift_right_logical(xi, 16)


def _halves_f32(xi):
    lo = plsc.bitcast(lax.shift_left(xi, 16), jnp.float32)
    hi = plsc.bitcast(xi & jnp.int32(-65536), jnp.float32)
    return lo, hi


def _sc_body(tab, gtnc, gen3, gre3, ncap_h, sco_out,
             tab_v, gtnc_v, gen_v, gre_v,
             tok_f, st_i, tsm_v, cnt_v, gsm_v, gcn_v,
             caps_v, gimg_v, rs_v, sco_v, ncap_v,
             sem0, sem1, sem2, sem3, sem4):
    wid = lax.axis_index("s") * 2 + lax.axis_index("c")

    cp_tab = pltpu.async_copy(tab, tab_v, sem0)
    cp_gtn = pltpu.async_copy(
        gtnc.at[pl.ds(pl.multiple_of(wid * GT_W, 8), GT_W)], gtnc_v, sem1)
    cp_gen = pltpu.async_copy(
        gen3.at[pl.ds(pl.multiple_of(wid * 3 * SEQ_W, 8), 3 * SEQ_W)],
        gen_v, sem2)
    cp_gre = pltpu.async_copy(
        gre3.at[pl.ds(pl.multiple_of(wid * 3 * SEQ_W, 8), 3 * SEQ_W)],
        gre_v, sem3)
    cp_nc = pltpu.async_copy(
        ncap_h.at[pl.ds(pl.multiple_of(wid * IMGS_W, 8), IMGS_W)],
        ncap_v, sem4)

    iota = lax.iota(jnp.int32, 16)
    iota20 = iota * SEQ
    zero16 = jnp.zeros((16,), jnp.float32)

    cp_tab.wait()

    @pl.loop(0, TOK_PAD // 256)
    def _(blk):
        for a in range(8):
            lo, hi = _halves_f32(
                tab_v[pl.ds(pl.multiple_of(blk * 128 + a * 16, 16), 16)])
            el = pl.multiple_of(blk * 256 + a * 16, 16)
            tok_f[pl.ds(el, 16)] = lo
            tok_f[pl.ds(pl.multiple_of(el + 128, 16), 16)] = hi

    @pl.loop(0, ST_PAD // 256)
    def _(blk):
        for a in range(8):
            lo, hi = _halves_i32(
                tab_v[pl.ds(pl.multiple_of(TOK_W + blk * 128 + a * 16, 16), 16)])
            el = pl.multiple_of(blk * 256 + a * 16, 16)
            st_i[pl.ds(el, 16)] = lo
            st_i[pl.ds(pl.multiple_of(el + 128, 16), 16)] = hi

    cp_gtn.wait()

    @pl.loop(0, GT_PAD // 256)
    def _(blk):
        for a in range(8):
            t0, t1 = _halves_i32(
                gtnc_v[pl.ds(pl.multiple_of(blk * 128 + a * 16, 16), 16)])
            el = blk * 256 + a * 16
            for h, tid in ((0, t0), (128, t1)):
                ts = plsc.load_gather(tok_f, [jnp.minimum(tid, TOK_PAD - 1)])
                valid = tid != 0
                off = pl.multiple_of(el + h, 16)
                gsm_v[pl.ds(off, 16)] = jnp.where(valid, ts, 0.0)
                gcn_v[pl.ds(off, 16)] = jnp.where(valid, 1.0, 0.0)

    @pl.loop(0, (IMGS_W * MAX_CAPS) // 16)
    def _(g):
        def t_body(t, carry):
            s, c = carry
            idx = iota20 + (g * 320 + t)
            s = s + plsc.load_gather(gsm_v, [idx])
            c = c + plsc.load_gather(gcn_v, [idx])
            return s, c

        s, c = lax.fori_loop(0, SEQ, t_body, (zero16, zero16))
        caps_v[pl.ds(pl.multiple_of(g * 16, 16), 16)] = s / jnp.maximum(c, 1.0)

    cp_nc.wait()
    nc0 = ncap_v[pl.ds(0, 16)]
    nc1 = ncap_v[pl.ds(16, 16)]
    for grp, ncg in ((0, nc0), (1, nc1)):
        imgoff = (grp * 16 + iota) * MAX_CAPS

        def c_body(c, gsum):
            cap = plsc.load_gather(caps_v, [imgoff + c])
            return gsum + jnp.where(c < ncg, cap, 0.0)

        gsum = lax.fori_loop(0, MAX_CAPS, c_body, zero16)
        gimg_v[pl.ds(grp * 16, 16)] = gsum / ncg.astype(jnp.float32)

    def seq_pass(src_v, out_base):
        @pl.loop(0, SEQ_EL // 256)
        def _(blk):
            for a in range(8):
                w = pl.multiple_of(blk * 128 + a * 16, 16)
                tx0, tx1 = _halves_i32(src_v[pl.ds(w, 16)])
                bn0, bn1 = _halves_i32(
                    src_v[pl.ds(pl.multiple_of(SEQ_W + w, 16), 16)])
                vi0, vi1 = _halves_i32(
                    src_v[pl.ds(pl.multiple_of(2 * SEQ_W + w, 16), 16)])
                el = out_base + blk * 256 + a * 16
                for h, tx, bn, vi in ((0, tx0, bn0, vi0), (128, tx1, bn1, vi1)):
                    mapped = plsc.load_gather(
                        st_i, [jnp.minimum(vi * 2 + bn - 1, ST_PAD - 1)])
                    res = jnp.where(tx < VOCAB, tx, mapped)
                    ts = plsc.load_gather(tok_f, [jnp.minimum(res, TOK_PAD - 1)])
                    valid = res != 0
                    off = pl.multiple_of(el + h, 16)
                    tsm_v[pl.ds(off, 16)] = jnp.where(valid, ts, 0.0)
                    cnt_v[pl.ds(off, 16)] = jnp.where(valid, 1.0, 0.0)

    cp_gen.wait()
    seq_pass(gen_v, 0)
    cp_gre.wait()
    seq_pass(gre_v, SEQ_EL)

    @pl.loop(0, (2 * ROWS_W) // 16)
    def _(g):
        def t_body(t, carry):
            s, c = carry
            idx = iota20 + (g * 320 + t)
            s = s + plsc.load_gather(tsm_v, [idx])
            c = c + plsc.load_gather(cnt_v, [idx])
            return s, c

        s, c = lax.fori_loop(0, SEQ, t_body, (zero16, zero16))
        rs_v[pl.ds(pl.multiple_of(g * 16, 16), 16)] = s / jnp.maximum(c, 1.0)

    iota4 = iota // 4
    for g in range(ROWS_W // 16):
        gen_s = rs_v[pl.ds(g * 16, 16)]
        gre_s = rs_v[pl.ds(ROWS_W + g * 16, 16)]
        gtv = plsc.load_gather(gimg_v, [iota4 + g * 4])
        sco_v[pl.ds(g * 16, 16)] = (gen_s - gre_s) * gtv

    pltpu.async_copy(sco_v, sco_out.at[pl.ds(pl.multiple_of(wid * SCO_W, 256),
                                             ROWS_W)], sem0).wait()


def _tc_body(s_ref, rew_ref, mean_ref):
    s = s_ref[...]
    rew_ref[...] = jnp.broadcast_to(s, (BATCH, SEQ))
    mean_ref[...] = jnp.full((1, 1), jnp.sum(s) * (1.0 / BATCH), jnp.float32)


def _packw(x):
    v = x.reshape(-1, 2, 128)
    return (v[:, 0, :] | (v[:, 1, :] << 16)).reshape(-1)


@jax.jit
def kernel(gen_txt_seq, gen_bn_seq, gen_vis_seq, greedy_txt_seq,
           greedy_bn_seq, greedy_vis_seq, gt_gts, ncap, st2towidx,
           token_scores):
    i32 = jnp.int32

    tok_bf = jnp.pad(token_scores.astype(jnp.bfloat16),
                     (0, TOK_PAD - token_scores.shape[0]))
    tok_h = lax.bitcast_convert_type(tok_bf, jnp.int16).astype(i32) & 0xFFFF
    st_h = jnp.pad(st2towidx.astype(i32), (0, ST_PAD - st2towidx.shape[0]))
    tab = jnp.concatenate([_packw(tok_h), _packw(st_h)])

    gt_p = jnp.pad(gt_gts.astype(i32).reshape(NW, GT_EL),
                   ((0, 0), (0, GT_PAD - GT_EL)))
    gtnc = _packw(gt_p.reshape(-1))

    def seq3(a, b, c):
        return jnp.concatenate(
            [_packw(x.astype(i32).reshape(-1)).reshape(NW, SEQ_W)
             for x in (a, b, c)], axis=1).reshape(-1)

    gen3 = seq3(gen_txt_seq, gen_bn_seq, gen_vis_seq)
    gre3 = seq3(greedy_txt_seq, greedy_bn_seq, greedy_vis_seq)

    mesh = plsc.VectorSubcoreMesh(core_axis_name="c", subcore_axis_name="s",
                                  num_cores=2, num_subcores=16)
    sc = pl.kernel(
        _sc_body,
        out_type=jax.ShapeDtypeStruct((NW * SCO_W,), jnp.float32),
        mesh=mesh,
        compiler_params=pltpu.CompilerParams(needs_layout_passes=False),
        scratch_types=[
            pltpu.VMEM((TAB_W,), jnp.int32),
            pltpu.VMEM((GT_W,), jnp.int32),
            pltpu.VMEM((3 * SEQ_W,), jnp.int32),
            pltpu.VMEM((3 * SEQ_W,), jnp.int32),
            pltpu.VMEM((TOK_PAD,), jnp.float32),
            pltpu.VMEM((ST_PAD,), jnp.int32),
            pltpu.VMEM((2 * SEQ_EL,), jnp.float32),
            pltpu.VMEM((2 * SEQ_EL,), jnp.float32),
            pltpu.VMEM((GT_PAD,), jnp.float32),
            pltpu.VMEM((GT_PAD,), jnp.float32),
            pltpu.VMEM((IMGS_W * MAX_CAPS,), jnp.float32),
            pltpu.VMEM((IMGS_W,), jnp.float32),
            pltpu.VMEM((2 * ROWS_W,), jnp.float32),
            pltpu.VMEM((ROWS_W,), jnp.float32),
            pltpu.VMEM((IMGS_W,), jnp.int32),
            pltpu.SemaphoreType.DMA,
            pltpu.SemaphoreType.DMA,
            pltpu.SemaphoreType.DMA,
            pltpu.SemaphoreType.DMA,
            pltpu.SemaphoreType.DMA,
        ],
    )
    scores = sc(tab, gtnc, gen3, gre3,
                ncap.astype(jnp.int32)).reshape(NW, SCO_W)[:, :ROWS_W]

    rewards, mean_arr = pl.pallas_call(
        _tc_body,
        out_shape=(jax.ShapeDtypeStruct((BATCH, SEQ), jnp.float32),
                   jax.ShapeDtypeStruct((1, 1), jnp.float32)),
    )(scores.reshape(BATCH, 1))

    return rewards, mean_arr[0, 0]


# speedup vs baseline: 8.3486x; 1.0197x over previous
import jax
import jax.numpy as jnp
from jax import lax
from jax.experimental import pallas as pl
from jax.experimental.pallas import tpu as pltpu
from jax.experimental.pallas import tpu_sc as plsc

VOCAB = 9487
BATCH = 4096
SEQ = 20
N_IMG = 1024
MAX_CAPS = 5

NW = 32
HALF_R = BATCH // 2
RW = 64
IW = 16
TOK_PAD = 9728
ST_PAD = 1024
TOK_H = TOK_PAD // 2
ST_H = ST_PAD // 2
TAB_W = TOK_H + ST_H


def _mo(x, n):
    return pl.multiple_of(x, n)


def _sc_body(txg, bng, vig, txr, bnr, vir, gtp, ncp, tab,
             rew_out, ps_out,
             tab_v, txg_v, bng_v, vig_v, txr_v, bnr_v, vir_v, gt_v, nc_v,
             tok_f, st_i, gimg_v, rs_v, rew_lo, rew_hi, acc_v,
             sem0, sem1, sem2, sem3, sem4):
    w = lax.axis_index("s") * 2 + lax.axis_index("c")
    rbase = _mo(w * RW, 8)
    ibase = _mo(w * IW, 8)

    cp_tab = pltpu.async_copy(tab, tab_v, sem0)
    cp_gt = pltpu.async_copy(gtp.at[pl.ds(ibase, IW)], gt_v, sem1)
    cp_nc = pltpu.async_copy(ncp.at[pl.ds(ibase, IW)], nc_v, sem4)
    cp_gen = [pltpu.async_copy(src.at[pl.ds(rbase, RW)], dst, sem2)
              for src, dst in ((txg, txg_v), (bng, bng_v), (vig, vig_v))]
    cp_gre = [pltpu.async_copy(src.at[pl.ds(rbase, RW)], dst, sem3)
              for src, dst in ((txr, txr_v), (bnr, bnr_v), (vir, vir_v))]

    iota = lax.iota(jnp.int32, 16)
    zero16 = jnp.zeros((16,), jnp.float32)
    one16 = jnp.ones((16,), jnp.float32)

    cp_tab.wait()

    @pl.loop(0, TOK_H // 16)
    def _(i):
        xi = tab_v[pl.ds(_mo(i * 16, 16), 16)]
        tok_f[pl.ds(_mo(i * 16, 16), 16)] = plsc.bitcast(
            lax.shift_left(xi, 16), jnp.float32)
        tok_f[pl.ds(_mo(TOK_H + i * 16, 16), 16)] = plsc.bitcast(
            xi & jnp.int32(-65536), jnp.float32)

    @pl.loop(0, ST_H // 16)
    def _(i):
        xi = tab_v[pl.ds(_mo(TOK_H + i * 16, 16), 16)]
        st_i[pl.ds(_mo(i * 16, 16), 16)] = xi & 0xFFFF
        st_i[pl.ds(_mo(ST_H + i * 16, 16), 16)] = lax.shift_right_logical(
            xi, 16)

    cp_gt.wait()
    cp_nc.wait()
    ncw = nc_v[...]
    nc_lo = ncw & 0xFFFF
    nc_hi = lax.shift_right_logical(ncw, 16)

    def cap_body(c, carry):
        glo, ghi = carry

        def t_body(t, inner):
            slo, clo, shi, chi = inner
            wv = plsc.load_gather(
                gt_v, [iota, jnp.full((16,), c * SEQ + t, jnp.int32)])
            tid_lo = wv & 0xFFFF
            tid_hi = lax.shift_right_logical(wv, 16)
            ts_lo = plsc.load_gather(tok_f, [tid_lo])
            ts_hi = plsc.load_gather(tok_f, [tid_hi])
            vlo = tid_lo != 0
            vhi = tid_hi != 0
            return (slo + jnp.where(vlo, ts_lo, 0.0),
                    clo + jnp.where(vlo, one16, 0.0),
                    shi + jnp.where(vhi, ts_hi, 0.0),
                    chi + jnp.where(vhi, one16, 0.0))

        slo, clo, shi, chi = lax.fori_loop(
            0, SEQ, t_body, (zero16, zero16, zero16, zero16))
        cap_lo = slo / jnp.maximum(clo, 1.0)
        cap_hi = shi / jnp.maximum(chi, 1.0)
        glo = glo + jnp.where(c < nc_lo, cap_lo, 0.0)
        ghi = ghi + jnp.where(c < nc_hi, cap_hi, 0.0)
        return glo, ghi

    glo, ghi = lax.fori_loop(0, MAX_CAPS, cap_body, (zero16, zero16))
    gimg_v[pl.ds(0, 16)] = glo / nc_lo.astype(jnp.float32)
    gimg_v[pl.ds(16, 16)] = ghi / nc_hi.astype(jnp.float32)

    def seq_rows(txv, bnv, viv, out0):
        for g in range(RW // 16):
            rows = iota + g * 16

            def t_body(t, inner):
                slo, clo, shi, chi = inner
                tcol = jnp.full((16,), t, jnp.int32)
                txw = plsc.load_gather(txv, [rows, tcol])
                bnw = plsc.load_gather(bnv, [rows, tcol])
                viw = plsc.load_gather(viv, [rows, tcol])
                for half in (0, 1):
                    if half == 0:
                        tx = txw & 0xFFFF
                        bn = bnw & 0xFFFF
                        vi = viw & 0xFFFF
                    else:
                        tx = lax.shift_right_logical(txw, 16)
                        bn = lax.shift_right_logical(bnw, 16)
                        vi = lax.shift_right_logical(viw, 16)
                    mapped = plsc.load_gather(st_i, [vi * 2 + bn - 1])
                    res = jnp.where(tx < VOCAB, tx, mapped)
                    ts = plsc.load_gather(tok_f, [res])
                    valid = res != 0
                    if half == 0:
                        slo = slo + jnp.where(valid, ts, 0.0)
                        clo = clo + jnp.where(valid, one16, 0.0)
                    else:
                        shi = shi + jnp.where(valid, ts, 0.0)
                        chi = chi + jnp.where(valid, one16, 0.0)
                return slo, clo, shi, chi

            slo, clo, shi, chi = lax.fori_loop(
                0, SEQ, t_body, (zero16, zero16, zero16, zero16))
            rs_v[pl.ds(out0 + g * 16, 16)] = slo / jnp.maximum(clo, 1.0)
            rs_v[pl.ds(out0 + RW + g * 16, 16)] = shi / jnp.maximum(chi, 1.0)

    for cp in cp_gen:
        cp.wait()
    seq_rows(txg_v, bng_v, vig_v, 0)
    for cp in cp_gre:
        cp.wait()
    seq_rows(txr_v, bnr_v, vir_v, 2 * RW)

    acc_v[...] = zero16
    iota4 = iota // 4
    for half, rew_v in ((0, rew_lo), (1, rew_hi)):
        for g in range(RW // 16):
            off = half * RW + g * 16
            gen_s = rs_v[pl.ds(off, 16)]
            gre_s = rs_v[pl.ds(2 * RW + off, 16)]
            gtv = plsc.load_gather(gimg_v, [iota4 + (half * IW + g * 4)])
            score = (gen_s - gre_s) * gtv
            acc_v[...] = acc_v[...] + score
            rows = iota + g * 16

            @pl.loop(0, SEQ)
            def _(t):
                plsc.store_scatter(rew_v,
                                   [rows, jnp.full((16,), t, jnp.int32)],
                                   score)

    pltpu.async_copy(rew_lo, rew_out.at[pl.ds(rbase, RW)], sem0).wait()
    pltpu.async_copy(rew_hi, rew_out.at[pl.ds(_mo(HALF_R + w * RW, 8), RW)],
                     sem0).wait()
    pltpu.async_copy(acc_v, ps_out.at[w], sem4).wait()


def _mean_body(ps_ref, o_ref):
    o_ref[...] = jnp.full((1, 1), jnp.sum(ps_ref[...]) * (1.0 / BATCH),
                          jnp.float32)


@jax.jit
def kernel(gen_txt_seq, gen_bn_seq, gen_vis_seq, greedy_txt_seq,
           greedy_bn_seq, greedy_vis_seq, gt_gts, ncap, st2towidx,
           token_scores):
    i32 = jnp.int32

    def packpair(x):
        a = x.astype(i32)
        n = a.shape[0] // 2
        return a[:n] | (a[n:] << 16)

    txg, bng, vig, txr, bnr, vir = (
        packpair(a) for a in (gen_txt_seq, gen_bn_seq, gen_vis_seq,
                              greedy_txt_seq, greedy_bn_seq, greedy_vis_seq))
    gtp = packpair(gt_gts).reshape(N_IMG // 2, MAX_CAPS * SEQ)
    ncp = packpair(ncap)

    tok_bf = jnp.pad(token_scores.astype(jnp.bfloat16),
                     (0, TOK_PAD - token_scores.shape[0]))
    th = lax.bitcast_convert_type(tok_bf, jnp.int16).astype(i32) & 0xFFFF
    st_h = jnp.pad(st2towidx.astype(i32), (0, ST_PAD - st2towidx.shape[0]))
    tab = jnp.concatenate([packpair(th), packpair(st_h)])

    mesh = plsc.VectorSubcoreMesh(core_axis_name="c", subcore_axis_name="s",
                                  num_cores=2, num_subcores=16)
    sc = pl.kernel(
        _sc_body,
        out_type=(jax.ShapeDtypeStruct((BATCH, SEQ), jnp.float32),
                  jax.ShapeDtypeStruct((NW, 16), jnp.float32)),
        mesh=mesh,
        compiler_params=pltpu.CompilerParams(needs_layout_passes=False),
        scratch_types=[
            pltpu.VMEM((TAB_W,), i32),
            pltpu.VMEM((RW, SEQ), i32),
            pltpu.VMEM((RW, SEQ), i32),
            pltpu.VMEM((RW, SEQ), i32),
            pltpu.VMEM((RW, SEQ), i32),
            pltpu.VMEM((RW, SEQ), i32),
            pltpu.VMEM((RW, SEQ), i32),
            pltpu.VMEM((IW, MAX_CAPS * SEQ), i32),
            pltpu.VMEM((IW,), i32),
            pltpu.VMEM((TOK_PAD,), jnp.float32),
            pltpu.VMEM((ST_PAD,), i32),
            pltpu.VMEM((2 * IW,), jnp.float32),
            pltpu.VMEM((4 * RW,), jnp.float32),
            pltpu.VMEM((RW, SEQ), jnp.float32),
            pltpu.VMEM((RW, SEQ), jnp.float32),
            pltpu.VMEM((16,), jnp.float32),
            pltpu.SemaphoreType.DMA,
            pltpu.SemaphoreType.DMA,
            pltpu.SemaphoreType.DMA,
            pltpu.SemaphoreType.DMA,
            pltpu.SemaphoreType.DMA,
        ],
    )
    rewards, psum = sc(txg, bng, vig, txr, bnr, vir, gtp, ncp, tab)

    mean_arr = pl.pallas_call(
        _mean_body,
        out_shape=jax.ShapeDtypeStruct((1, 1), jnp.float32),
    )(psum)

    return rewards, mean_arr[0, 0]


# speedup vs baseline: 9.8950x; 1.1852x over previous
import jax
import jax.numpy as jnp
from jax import lax
from jax.experimental import pallas as pl
from jax.experimental.pallas import tpu as pltpu
from jax.experimental.pallas import tpu_sc as plsc

VOCAB = 9487
BATCH = 4096
SEQ = 20
N_IMG = 1024
MAX_CAPS = 5

NW = 32
ROWS_W = BATCH // NW
IMGS_W = N_IMG // NW
SEQ_ELEMS_W = ROWS_W * SEQ
GT_ELEMS_W = IMGS_W * MAX_CAPS * SEQ
TOK_PAD = 9488
ST_PAD = 1024


def _sc_body(gtx, gbn, gvi, rtx, rbn, rvi, gt, ncap_h, st, tok,
             rew_out, ps_out,
             tok_v, st_v, gtx_v, gbn_v, gvi_v, rtx_v, rbn_v, rvi_v,
             gt_v, ncap_v, gimg_v, rew_v, acc_v, sem):
    wid = lax.axis_index("s") * 2 + lax.axis_index("c")
    sbase = wid * SEQ_ELEMS_W

    copies = [
        pltpu.async_copy(tok, tok_v, sem),
        pltpu.async_copy(st, st_v, sem),
        pltpu.async_copy(gtx.at[pl.ds(sbase, SEQ_ELEMS_W)], gtx_v, sem),
        pltpu.async_copy(gbn.at[pl.ds(sbase, SEQ_ELEMS_W)], gbn_v, sem),
        pltpu.async_copy(gvi.at[pl.ds(sbase, SEQ_ELEMS_W)], gvi_v, sem),
        pltpu.async_copy(rtx.at[pl.ds(sbase, SEQ_ELEMS_W)], rtx_v, sem),
        pltpu.async_copy(rbn.at[pl.ds(sbase, SEQ_ELEMS_W)], rbn_v, sem),
        pltpu.async_copy(rvi.at[pl.ds(sbase, SEQ_ELEMS_W)], rvi_v, sem),
        pltpu.async_copy(gt.at[pl.ds(wid * GT_ELEMS_W, GT_ELEMS_W)], gt_v, sem),
        pltpu.async_copy(ncap_h.at[pl.ds(wid * IMGS_W, IMGS_W)], ncap_v, sem),
    ]
    for c in copies:
        c.wait()

    iota = lax.iota(jnp.int32, 16)
    zero16 = jnp.zeros((16,), jnp.float32)

    for grp in range(IMGS_W // 16):
        ncap_i = ncap_v[pl.ds(grp * 16, 16)]
        ioff = iota * (MAX_CAPS * SEQ) + grp * 16 * (MAX_CAPS * SEQ)

        def cap_body(c, gsum):
            def t_body(t, carry):
                s, cnt = carry
                tid = plsc.load_gather(gt_v, [ioff + (c * SEQ + t)])
                ts = plsc.load_gather(tok_v, [tid])
                valid = tid != 0
                s = s + jnp.where(valid, ts, 0.0)
                cnt = cnt + jnp.where(valid, 1.0, 0.0)
                return s, cnt

            s, cnt = lax.fori_loop(0, SEQ, t_body, (zero16, zero16))
            cap_score = s / jnp.maximum(cnt, 1.0)
            return gsum + jnp.where(c < ncap_i, cap_score, 0.0)

        gsum = lax.fori_loop(0, MAX_CAPS, cap_body, zero16)
        gimg_v[pl.ds(grp * 16, 16)] = gsum / ncap_i.astype(jnp.float32)

    acc_v[...] = zero16

    @pl.loop(0, ROWS_W // 16)
    def _(g):
        rbase = g * 16 * SEQ

        def seq_score(txt_v, bn_v, vis_v):
            def t_body(t, carry):
                s, cnt = carry
                idx = iota * SEQ + rbase + t
                txt = plsc.load_gather(txt_v, [idx])
                bn = plsc.load_gather(bn_v, [idx])
                vis = plsc.load_gather(vis_v, [idx])
                mapped = plsc.load_gather(st_v, [vis * 2 + bn - 1])
                res = jnp.where(txt < VOCAB, txt, mapped)
                ts = plsc.load_gather(tok_v, [res])
                valid = res != 0
                s = s + jnp.where(valid, ts, 0.0)
                cnt = cnt + jnp.where(valid, 1.0, 0.0)
                return s, cnt

            s, cnt = lax.fori_loop(0, SEQ, t_body, (zero16, zero16))
            return s / jnp.maximum(cnt, 1.0)

        gen_s = seq_score(gtx_v, gbn_v, gvi_v)
        gre_s = seq_score(rtx_v, rbn_v, rvi_v)
        gtv = plsc.load_gather(gimg_v, [(g * 16 + iota) // 4])
        score = (gen_s - gre_s) * gtv
        acc_v[...] = acc_v[...] + score

        @pl.loop(0, SEQ)
        def _(t):
            plsc.store_scatter(rew_v, [iota * SEQ + rbase + t], score)

    pltpu.async_copy(rew_v, rew_out.at[pl.ds(sbase, SEQ_ELEMS_W)], sem).wait()
    pltpu.async_copy(acc_v, ps_out.at[wid], sem).wait()


def _mean_body(ps_ref, o_ref):
    o_ref[...] = jnp.full((1, 1), jnp.sum(ps_ref[...]) * (1.0 / BATCH),
                          jnp.float32)


@jax.jit
def kernel(gen_txt_seq, gen_bn_seq, gen_vis_seq, greedy_txt_seq,
           greedy_bn_seq, greedy_vis_seq, gt_gts, ncap, st2towidx,
           token_scores):
    i32 = jnp.int32
    gtx = gen_txt_seq.astype(i32).reshape(-1)
    gbn = gen_bn_seq.astype(i32).reshape(-1)
    gvi = gen_vis_seq.astype(i32).reshape(-1)
    rtx = greedy_txt_seq.astype(i32).reshape(-1)
    rbn = greedy_bn_seq.astype(i32).reshape(-1)
    rvi = greedy_vis_seq.astype(i32).reshape(-1)
    gt = gt_gts.astype(i32).reshape(-1)
    ncap_i = ncap.astype(i32)
    st = jnp.pad(st2towidx.astype(i32), (0, ST_PAD - st2towidx.shape[0]))
    tok = jnp.pad(token_scores.astype(jnp.float32),
                  (0, TOK_PAD - token_scores.shape[0]))

    mesh = plsc.VectorSubcoreMesh(core_axis_name="c", subcore_axis_name="s",
                                  num_cores=2, num_subcores=16)
    sc = pl.kernel(
        _sc_body,
        out_type=(jax.ShapeDtypeStruct((BATCH * SEQ,), jnp.float32),
                  jax.ShapeDtypeStruct((NW, 16), jnp.float32)),
        mesh=mesh,
        compiler_params=pltpu.CompilerParams(needs_layout_passes=False),
        scratch_types=[
            pltpu.VMEM((TOK_PAD,), jnp.float32),
            pltpu.VMEM((ST_PAD,), i32),
            pltpu.VMEM((SEQ_ELEMS_W,), i32),
            pltpu.VMEM((SEQ_ELEMS_W,), i32),
            pltpu.VMEM((SEQ_ELEMS_W,), i32),
            pltpu.VMEM((SEQ_ELEMS_W,), i32),
            pltpu.VMEM((SEQ_ELEMS_W,), i32),
            pltpu.VMEM((SEQ_ELEMS_W,), i32),
            pltpu.VMEM((GT_ELEMS_W,), i32),
            pltpu.VMEM((IMGS_W,), i32),
            pltpu.VMEM((IMGS_W,), jnp.float32),
            pltpu.VMEM((SEQ_ELEMS_W,), jnp.float32),
            pltpu.VMEM((16,), jnp.float32),
            pltpu.SemaphoreType.DMA,
        ],
    )
    rew_flat, psum = sc(gtx, gbn, gvi, rtx, rbn, rvi, gt, ncap_i, st, tok)

    mean_arr = pl.pallas_call(
        _mean_body,
        out_shape=jax.ShapeDtypeStruct((1, 1), jnp.float32),
    )(psum)

    return rew_flat.reshape(BATCH, SEQ), mean_arr[0, 0]
